# baseline (speedup 1.0000x reference)
import functools

import jax
import jax.numpy as jnp
from jax import lax
from jax.experimental import pallas as pl
from jax.experimental.pallas import tpu as pltpu
from jax.experimental.pallas import tpu_sc as plsc

B, S, D, V = 4096, 200, 128, 100000
NC, NS, L = 2, 16, 16
NW = NC * NS
BPW = B // NW
NB = 4
SPLIT = 104


def _emb_body(ids_hbm, pos_hbm, table_hbm, out_hbm,
              pos_v, i0, i1, i2, i3, r0, r1, r2, r3,
              is0, is1, is2, is3, gs0, gs1, gs2, gs3,
              os0, os1, os2, os3):
    idxs = (i0, i1, i2, i3)
    rb = (r0, r1, r2, r3)
    isems = (is0, is1, is2, is3)
    gsems = (gs0, gs1, gs2, gs3)
    osems = (os0, os1, os2, os3)

    wid = lax.axis_index("s") * NC + lax.axis_index("c")
    b0 = wid * BPW
    pltpu.sync_copy(pos_hbm.at[0], pos_v)

    def idx_start(r, b):
        pltpu.async_copy(ids_hbm.at[b], idxs[r], isems[r])

    def idx_wait(r):
        pltpu.make_async_copy(ids_hbm.at[b0], idxs[r], isems[r]).wait()

    def gather_start(r):
        pltpu.async_copy(table_hbm.at[idxs[r].at[pl.ds(0, SPLIT)]],
                         rb[r].at[pl.ds(0, SPLIT)], gsems[r])
        pltpu.async_copy(table_hbm.at[idxs[r].at[pl.ds(SPLIT, S - SPLIT)]],
                         rb[r].at[pl.ds(SPLIT, S - SPLIT)], gsems[r])

    def gather_wait(r):
        pltpu.make_async_copy(table_hbm.at[idxs[r].at[pl.ds(0, SPLIT)]],
                              rb[r].at[pl.ds(0, SPLIT)], gsems[r]).wait()
        pltpu.make_async_copy(table_hbm.at[idxs[r].at[pl.ds(SPLIT, S - SPLIT)]],
                              rb[r].at[pl.ds(SPLIT, S - SPLIT)], gsems[r]).wait()

    def out_start(r, b):
        pltpu.async_copy(rb[r], out_hbm.at[b], osems[r])

    def out_wait(r):
        pltpu.make_async_copy(rb[r], out_hbm.at[b0], osems[r]).wait()

    idx_start(0, b0)
    idx_start(1, b0 + 1)
    idx_start(2, b0 + 2)
    idx_wait(0)
    gather_start(0)
    idx_wait(1)
    gather_start(1)

    def iter_body(i, r):
        rg = (r + 2) % NB
        ri = (r + 3) % NB

        def prefetch_idx():
            idx_start(ri, b0 + i + 3)
        pl.when(i + 3 < BPW)(prefetch_idx)

        def start_next_gather():
            def drain_out():
                out_wait(rg)
            pl.when(i >= 2)(drain_out)
            idx_wait(rg)
            gather_start(rg)
        pl.when(i + 2 < BPW)(start_next_gather)

        gather_wait(r)

        @plsc.parallel_loop(0, S, unroll=4)
        def addrow(row):
            for p in range(D // L):
                sl = pl.ds(p * L, L)
                rb[r][row, sl] = rb[r][row, sl] + pos_v[row, sl]

        out_start(r, b0 + i)

    def outer(g, c):
        for r in range(NB):
            iter_body(NB * g + r, r)
        return c

    lax.fori_loop(0, BPW // NB, outer, 0)
    for r in range(NB):
        out_wait(r)


@jax.jit
def kernel(input_ids, table, pos_embed):
    mesh = plsc.VectorSubcoreMesh(core_axis_name="c", subcore_axis_name="s")
    return pl.kernel(
        _emb_body,
        mesh=mesh,
        out_type=jax.ShapeDtypeStruct((B, S, D), jnp.float32),
        scratch_types=(
            [pltpu.VMEM((S, D), jnp.float32)]
            + [pltpu.VMEM((S,), jnp.int32)] * NB
            + [pltpu.VMEM((S, D), jnp.float32)] * NB
            + [pltpu.SemaphoreType.DMA] * (3 * NB)
        ),
    )(input_ids, pos_embed, table)
